# baseline (speedup 1.0000x reference)
import functools

import jax
import jax.numpy as jnp
from jax import lax
from jax.experimental import pallas as pl
from jax.experimental.pallas import tpu as pltpu
from jax.experimental.pallas import tpu_sc as plsc

_LN_EPS = 1e-5
_B, _C, _N = 32, 96, 16384
_K = 4096
_CSC = 56
_CPAD = 64
_CTC = _C - _CSC
_TCBLK = 8

_L = 16
_SEG = _N // _L
_NB1 = 512
_HS = 512
_LEVELS = [(18, 32), (13, 32), (8, 32), (4, 16), (0, 16)]
_NCORES = 2


def _sc_body(fm_hbm, out_hbm, row0, row1, lista, listb, hist, salbuf,
             sem0, sem1):
    iota = lax.iota(jnp.int32, _L)
    lane_seg = iota * _SEG
    zero16 = jnp.zeros((_L,), jnp.int32)
    zero16f = jnp.zeros((_L,), jnp.float32)
    ones16 = jnp.ones((_L,), jnp.int32)
    lane0 = iota == 0

    b = lax.axis_index("s") * _NCORES + lax.axis_index("c")

    def bcast_i(x):
        return lax.broadcast_in_dim(x, (_L,), ())

    def bcast_f(x):
        return lax.broadcast_in_dim(x, (_L,), ())

    def key_of(s):
        return s ^ ((s >> 31) & jnp.int32(0x7FFFFFFF))

    def select_bucket(nb, k_cur, total):
        carry = jnp.int32(0)
        na_v = zero16
        nge_v = zero16
        for ci in range(nb // _L):
            acc = hist[pl.ds(ci * _L, _L)]
            hist[pl.ds(ci * _L, _L)] = zero16
            local = plsc.cumsum(acc)
            cum = local + carry
            suffix = total - cum + acc
            ge = suffix >= k_cur
            nge_v = nge_v + ge.astype(jnp.int32)
            na_v = na_v + jnp.where(ge, 0, acc)
            carry = carry + local[_L - 1]
        return jnp.sum(nge_v) - 1, jnp.sum(na_v)

    def level_list(src, dst, shift, nb, lane_cnt, k_cur, sa):
        total = jnp.sum(lane_cnt)
        mx = jnp.max(lane_cnt)
        mask_b = jnp.int32(nb - 1)

        @plsc.parallel_loop(0, mx, unroll=4)
        def _(j):
            key = plsc.load_gather(src, [lane_seg + j])
            valid = j < lane_cnt
            bucket = (key >> shift) & mask_b
            plsc.addupdate_scatter(hist, [bucket], ones16, mask=valid)

        bstar, na = select_bucket(nb, k_cur, total)

        def cbody(j, st):
            cnt, sa = st
            key = plsc.load_gather(src, [lane_seg + j])
            valid = j < lane_cnt
            v = plsc.bitcast(key_of(key), jnp.float32)
            bucket = (key >> shift) & mask_b
            gt = jnp.logical_and(bucket > bstar, valid)
            sa = sa + jnp.where(gt, v, 0.0)
            eq = jnp.logical_and(bucket == bstar, valid)
            plsc.store_scatter(dst, [lane_seg + cnt], key, mask=eq)
            cnt = cnt + eq.astype(jnp.int32)
            return cnt, sa

        cnt, sa = plsc.parallel_loop(0, mx, carry=(zero16, sa),
                                     unroll=4)(cbody)
        return bstar, cnt, k_cur - na, sa

    def select_row(row_ref, c):
        @plsc.parallel_loop(0, _N, step=_L, unroll=8)
        def _(j):
            x = row_ref[pl.ds(j, _L)]
            s = plsc.bitcast(x, jnp.int32)
            bucket = (key_of(s) >> 23) + 256
            plsc.addupdate_scatter(hist, [bucket], ones16)

        bstar1, na1 = select_bucket(_NB1, jnp.int32(_K), jnp.int32(_N))

        def c1(j, st):
            cnt, sa = st
            x = row_ref[pl.ds(j, _L)]
            s = plsc.bitcast(x, jnp.int32)
            bucket = (key_of(s) >> 23) + 256
            gt = bucket > bstar1
            sa = sa + jnp.where(gt, x, 0.0)
            eq = bucket == bstar1
            plsc.store_scatter(lista, [lane_seg + cnt], key_of(s), mask=eq)
            cnt = cnt + eq.astype(jnp.int32)
            return cnt, sa

        cnt, sa = plsc.parallel_loop(0, _N, step=_L,
                                     carry=(zero16, zero16f),
                                     unroll=8)(c1)
        kc = jnp.int32(_K) - na1
        k_acc = (bstar1 - 256) << 23
        src, dst = lista, listb
        for shift, nb in _LEVELS:
            bst, cnt, kc, sa = level_list(src, dst, shift, nb, cnt, kc, sa)
            k_acc = k_acc + (bst << shift)
            src, dst = dst, src

        vk = plsc.bitcast(key_of(bcast_i(k_acc)), jnp.float32)
        ties = jnp.where(lane0, bcast_f(kc.astype(jnp.float32)) * vk, 0.0)
        sal = jnp.sum((sa + ties) * (1.0 / _K))
        plsc.store_scatter(salbuf, [bcast_i(c)], bcast_f(sal), mask=lane0)

    pltpu.make_async_copy(fm_hbm.at[b, 0], row0, sem0).start()
    for ci in range(_HS // _L):
        hist[pl.ds(ci * _L, _L)] = zero16
    salbuf[pl.ds(_CPAD - _L, _L)] = zero16f

    def rowpair(i, _):
        c0 = i * 2
        pltpu.make_async_copy(fm_hbm.at[b, c0], row0, sem0).wait()
        cp1 = pltpu.make_async_copy(fm_hbm.at[b, c0 + 1], row1, sem1)
        cp1.start()
        select_row(row0, c0)
        cp1.wait()

        @pl.when(i < _CSC // 2 - 1)
        def _():
            pltpu.make_async_copy(fm_hbm.at[b, c0 + 2], row0, sem0).start()

        select_row(row1, c0 + 1)
        return 0

    lax.fori_loop(0, _CSC // 2, rowpair, 0)
    pltpu.sync_copy(salbuf, out_hbm.at[b])


_sc_salient = functools.partial(
    pl.kernel,
    mesh=plsc.VectorSubcoreMesh(core_axis_name="c", subcore_axis_name="s"),
    out_type=jax.ShapeDtypeStruct((_B, _CPAD), jnp.float32),
    compiler_params=pltpu.CompilerParams(needs_layout_passes=False),
    scratch_types=[
        pltpu.VMEM((_N,), jnp.float32),
        pltpu.VMEM((_N,), jnp.float32),
        pltpu.VMEM((_N,), jnp.int32),
        pltpu.VMEM((_N,), jnp.int32),
        pltpu.VMEM((_HS,), jnp.int32),
        pltpu.VMEM((_CPAD,), jnp.float32),
        pltpu.SemaphoreType.DMA,
        pltpu.SemaphoreType.DMA,
    ],
)(_sc_body)


def _to_ukey(x):
    u = jax.lax.bitcast_convert_type(x, jnp.uint32)
    topbit = jnp.uint32(0x80000000)
    return jnp.where(u >= topbit, ~u, u | topbit)


def _from_ukey(p):
    topbit = jnp.uint32(0x80000000)
    u = jnp.where(p >= topbit, p ^ topbit, ~p)
    return jax.lax.bitcast_convert_type(u, jnp.float32)


def _tc_salient_body(x_ref, out_ref):
    x = x_ref[0]
    ukey = _to_ukey(x)
    kk = jnp.int32(_K)

    def step(i, p):
        bit = jnp.uint32(31) - jnp.uint32(i)
        cand = p | (jnp.uint32(1) << bit)
        cnt = jnp.sum((ukey >= cand).astype(jnp.int32), axis=-1,
                      keepdims=True)
        return jnp.where(cnt >= kk, cand, p)

    p0 = jnp.zeros((x.shape[0], 1), dtype=jnp.uint32)
    p = jax.lax.fori_loop(0, 32, step, p0)

    gt = ukey > p
    cnt_gt = jnp.sum(gt.astype(jnp.int32), axis=-1)
    s_gt = jnp.sum(jnp.where(gt, x, 0.0), axis=-1)
    vk = _from_ukey(p[:, 0])
    total = s_gt + (kk - cnt_gt).astype(jnp.float32) * vk
    out_ref[0, 0, 0] = total * (1.0 / _K)


def _fusion_body(pooled_ref, salient_ref, ln1_w_ref, ln1_b_ref,
                 gs_ref, gb_ref, rs_ref, ln2_w_ref, ln2_b_ref, out_ref):
    def ln(v, w, b):
        mu = jnp.mean(v, axis=-1, keepdims=True)
        var = jnp.mean((v - mu) ** 2, axis=-1, keepdims=True)
        return (v - mu) * jax.lax.rsqrt(var + _LN_EPS) * w + b

    pooled = pooled_ref[...]
    salient = ln(salient_ref[...], ln1_w_ref[...], ln1_b_ref[...])
    delta = salient - pooled
    gate = jax.nn.sigmoid(gs_ref[...] * salient + gb_ref[...])
    fused = pooled + rs_ref[...] * gate * delta
    out_ref[...] = ln(fused, ln2_w_ref[...], ln2_b_ref[...])


def kernel(pooled, feature_map, ln1_w, ln1_b, gate_scale, gate_bias,
           residual_scale, ln2_w, ln2_b):
    fm = feature_map.astype(jnp.float32).reshape(_B, _C, _N)

    sal_sc = _sc_salient(fm)[:, :_CSC]

    sal_tc = pl.pallas_call(
        _tc_salient_body,
        grid=(_B, _CTC // _TCBLK),
        in_specs=[pl.BlockSpec((1, _TCBLK, _N),
                               lambda b, j: (b, _CSC // _TCBLK + j, 0))],
        out_specs=pl.BlockSpec((1, 1, 1, _TCBLK), lambda b, j: (b, j, 0, 0)),
        out_shape=jax.ShapeDtypeStruct((_B, _CTC // _TCBLK, 1, _TCBLK),
                                       jnp.float32),
    )(fm).reshape(_B, _CTC)

    salient = jnp.concatenate([sal_sc, sal_tc], axis=1)
    params = [p.reshape(1, _C) for p in
              (ln1_w, ln1_b, gate_scale, gate_bias, residual_scale,
               ln2_w, ln2_b)]
    out = pl.pallas_call(
        _fusion_body,
        out_shape=jax.ShapeDtypeStruct((_B, _C), jnp.float32),
    )(pooled.astype(jnp.float32), salient, *params)
    return out.astype(pooled.dtype)


# speedup vs baseline: 1.2639x; 1.2639x over previous
import functools

import jax
import jax.numpy as jnp
from jax import lax
from jax.experimental import pallas as pl
from jax.experimental.pallas import tpu as pltpu
from jax.experimental.pallas import tpu_sc as plsc

_LN_EPS = 1e-5
_B, _C, _N = 32, 96, 16384
_K = 4096
_L = 16
_SEG = _N // _L
_NB1 = 512
_HS = 512
_LEVELS = [(14, 512), (5, 512), (0, 32)]
_NCORES = 2


def _sc_body(pooled_hbm, fm_hbm, params_hbm, out_hbm,
             row0, row1, lista, listb, hist,
             salbuf, pooledbuf, parbuf, outbuf, sem0, sem1):
    iota = lax.iota(jnp.int32, _L)
    lane_seg = iota * _SEG
    zero16 = jnp.zeros((_L,), jnp.int32)
    zero16f = jnp.zeros((_L,), jnp.float32)
    ones16 = jnp.ones((_L,), jnp.int32)
    lane0 = iota == 0

    b = lax.axis_index("s") * _NCORES + lax.axis_index("c")

    def bcast_i(x):
        return lax.broadcast_in_dim(x, (_L,), ())

    def bcast_f(x):
        return lax.broadcast_in_dim(x, (_L,), ())

    def key_of(s):
        return s ^ ((s >> 31) & jnp.int32(0x7FFFFFFF))

    def select_bucket(nb, k_cur, total):
        carry = jnp.int32(0)
        na_v = zero16
        nge_v = zero16
        for ci in range(nb // _L):
            acc = hist[pl.ds(ci * _L, _L)]
            hist[pl.ds(ci * _L, _L)] = zero16
            local = plsc.cumsum(acc)
            cum = local + carry
            suffix = total - cum + acc
            ge = suffix >= k_cur
            nge_v = nge_v + ge.astype(jnp.int32)
            na_v = na_v + jnp.where(ge, 0, acc)
            carry = carry + local[_L - 1]
        return jnp.sum(nge_v) - 1, jnp.sum(na_v)

    def level_list(src, dst, shift, nb, lane_cnt, k_cur, sa, k_acc):
        total = jnp.sum(lane_cnt)
        mx = jnp.max(lane_cnt)
        mask_b = jnp.int32(nb - 1)

        @plsc.parallel_loop(0, mx, unroll=8)
        def _(j):
            key = plsc.load_gather(src, [lane_seg + j])
            valid = j < lane_cnt
            bucket = (key >> shift) & mask_b
            plsc.addupdate_scatter(hist, [bucket], ones16, mask=valid)

        bstar, na = select_bucket(nb, k_cur, total)
        lo = k_acc + (bstar << shift)
        hi = lo + jnp.int32((1 << shift) - 1)

        def cbody(j, st):
            cnt, sa = st
            key = plsc.load_gather(src, [lane_seg + j])
            valid = j < lane_cnt
            v = plsc.bitcast(key_of(key), jnp.float32)
            gt = jnp.logical_and(key > hi, valid)
            sa = sa + jnp.where(gt, v, 0.0)
            eq = jnp.logical_and(jnp.logical_and(key >= lo, key <= hi),
                                 valid)
            plsc.store_scatter(dst, [lane_seg + cnt], key, mask=eq)
            cnt = cnt + eq.astype(jnp.int32)
            return cnt, sa

        cnt, sa = plsc.parallel_loop(0, mx, carry=(zero16, sa),
                                     unroll=8)(cbody)
        return bstar, cnt, k_cur - na, sa

    def select_row(row_ref, c):
        @plsc.parallel_loop(0, _N, step=_L, unroll=8)
        def _(j):
            x = row_ref[pl.ds(j, _L)]
            s = plsc.bitcast(x, jnp.int32)
            bucket = (key_of(s) >> 23) + 256
            plsc.addupdate_scatter(hist, [bucket], ones16)

        bstar1, na1 = select_bucket(_NB1, jnp.int32(_K), jnp.int32(_N))

        def c1(j, st):
            cnt, sa = st
            x = row_ref[pl.ds(j, _L)]
            s = plsc.bitcast(x, jnp.int32)
            bucket = (key_of(s) >> 23) + 256
            gt = bucket > bstar1
            sa = sa + jnp.where(gt, x, 0.0)
            eq = bucket == bstar1
            plsc.store_scatter(lista, [lane_seg + cnt], key_of(s), mask=eq)
            cnt = cnt + eq.astype(jnp.int32)
            return cnt, sa

        cnt, sa = plsc.parallel_loop(0, _N, step=_L,
                                     carry=(zero16, zero16f),
                                     unroll=8)(c1)
        kc = jnp.int32(_K) - na1
        k_acc = (bstar1 - 256) << 23
        src, dst = lista, listb
        for shift, nb in _LEVELS:
            bst, cnt, kc, sa = level_list(src, dst, shift, nb, cnt, kc, sa,
                                          k_acc)
            k_acc = k_acc + (bst << shift)
            src, dst = dst, src

        vk = plsc.bitcast(key_of(bcast_i(k_acc)), jnp.float32)
        ties = jnp.where(lane0, bcast_f(kc.astype(jnp.float32)) * vk, 0.0)
        sal = jnp.sum((sa + ties) * (1.0 / _K))
        plsc.store_scatter(salbuf, [bcast_i(c)], bcast_f(sal), mask=lane0)

    pltpu.make_async_copy(fm_hbm.at[b, 0], row0, sem0).start()
    for ci in range(_HS // _L):
        hist[pl.ds(ci * _L, _L)] = zero16
    pltpu.sync_copy(pooled_hbm.at[b], pooledbuf)
    pltpu.sync_copy(params_hbm, parbuf)

    def rowpair(i, _):
        c0 = i * 2
        pltpu.make_async_copy(fm_hbm.at[b, c0], row0, sem0).wait()
        cp1 = pltpu.make_async_copy(fm_hbm.at[b, c0 + 1], row1, sem1)
        cp1.start()
        select_row(row0, c0)
        cp1.wait()

        @pl.when(i < _C // 2 - 1)
        def _():
            pltpu.make_async_copy(fm_hbm.at[b, c0 + 2], row0, sem0).start()

        select_row(row1, c0 + 1)
        return 0

    lax.fori_loop(0, _C // 2, rowpair, 0)

    def rsqrt16(v):
        i = plsc.bitcast(v, jnp.int32)
        y = plsc.bitcast(jnp.int32(0x5F3759DF) - (i >> 1), jnp.float32)
        for _ in range(4):
            y = y * (1.5 - 0.5 * v * y * y)
        return y

    nch = _C // _L

    def stats(ref):
        accv = zero16f
        for ci in range(nch):
            accv = accv + ref[pl.ds(ci * _L, _L)]
        mean = bcast_f(jnp.sum(accv) * (1.0 / _C))
        varv = zero16f
        for ci in range(nch):
            d = ref[pl.ds(ci * _L, _L)] - mean
            varv = varv + d * d
        rstd = rsqrt16(bcast_f(jnp.sum(varv) * (1.0 / _C)) + _LN_EPS)
        return mean, rstd

    mean1, rstd1 = stats(salbuf)
    for ci in range(nch):
        sl = pl.ds(ci * _L, _L)
        w1 = parbuf[pl.ds(0 * _C + ci * _L, _L)]
        b1 = parbuf[pl.ds(1 * _C + ci * _L, _L)]
        gs = parbuf[pl.ds(2 * _C + ci * _L, _L)]
        gb = parbuf[pl.ds(3 * _C + ci * _L, _L)]
        rs = parbuf[pl.ds(4 * _C + ci * _L, _L)]
        sal = (salbuf[sl] - mean1) * rstd1 * w1 + b1
        pld = pooledbuf[sl]
        gate = 1.0 / (1.0 + jnp.exp(-(gs * sal + gb)))
        outbuf[sl] = pld + rs * gate * (sal - pld)

    mean2, rstd2 = stats(outbuf)
    for ci in range(nch):
        sl = pl.ds(ci * _L, _L)
        w2 = parbuf[pl.ds(5 * _C + ci * _L, _L)]
        b2 = parbuf[pl.ds(6 * _C + ci * _L, _L)]
        outbuf[sl] = (outbuf[sl] - mean2) * rstd2 * w2 + b2

    pltpu.sync_copy(outbuf, out_hbm.at[b])


_sc_kernel = functools.partial(
    pl.kernel,
    mesh=plsc.VectorSubcoreMesh(core_axis_name="c", subcore_axis_name="s"),
    out_type=jax.ShapeDtypeStruct((_B, _C), jnp.float32),
    compiler_params=pltpu.CompilerParams(needs_layout_passes=False),
    scratch_types=[
        pltpu.VMEM((_N,), jnp.float32),
        pltpu.VMEM((_N,), jnp.float32),
        pltpu.VMEM((_N,), jnp.int32),
        pltpu.VMEM((_N,), jnp.int32),
        pltpu.VMEM((_HS,), jnp.int32),
        pltpu.VMEM((_C,), jnp.float32),
        pltpu.VMEM((_C,), jnp.float32),
        pltpu.VMEM((7 * _C,), jnp.float32),
        pltpu.VMEM((_C,), jnp.float32),
        pltpu.SemaphoreType.DMA,
        pltpu.SemaphoreType.DMA,
    ],
)(_sc_body)


def kernel(pooled, feature_map, ln1_w, ln1_b, gate_scale, gate_bias,
           residual_scale, ln2_w, ln2_b):
    fm = feature_map.astype(jnp.float32).reshape(_B, _C, _N)
    params = jnp.concatenate([
        ln1_w, ln1_b, gate_scale, gate_bias, residual_scale, ln2_w, ln2_b,
    ]).astype(jnp.float32)
    out = _sc_kernel(pooled.astype(jnp.float32), fm, params)
    return out.astype(pooled.dtype)


# speedup vs baseline: 1.4441x; 1.1426x over previous
import functools

import jax
import jax.numpy as jnp
from jax import lax
from jax.experimental import pallas as pl
from jax.experimental.pallas import tpu as pltpu
from jax.experimental.pallas import tpu_sc as plsc

_LN_EPS = 1e-5
_B, _C, _N = 32, 96, 16384
_K = 4096
_L = 16
_SEG = _N // _L
_NB1 = 512
_HS = 512
_LEVELS = [(14, 512), (5, 512), (0, 32)]
_NCORES = 2


def _sc_body(pooled_hbm, fm_hbm, params_hbm, out_hbm,
             row0, row1, lista, listb, hist, sumh,
             salbuf, pooledbuf, parbuf, outbuf, sem0, sem1):
    iota = lax.iota(jnp.int32, _L)
    lane_seg = iota * _SEG
    zero16 = jnp.zeros((_L,), jnp.int32)
    zero16f = jnp.zeros((_L,), jnp.float32)
    ones16 = jnp.ones((_L,), jnp.int32)
    lane0 = iota == 0

    b = lax.axis_index("s") * _NCORES + lax.axis_index("c")

    def bcast_i(x):
        return lax.broadcast_in_dim(x, (_L,), ())

    def bcast_f(x):
        return lax.broadcast_in_dim(x, (_L,), ())

    def key_of(s):
        return s ^ ((s >> 31) & jnp.int32(0x7FFFFFFF))

    def select_bucket(nb, k_cur, total):
        carry = jnp.int32(0)
        na_v = zero16
        nge_v = zero16
        for ci in range(nb // _L):
            acc = hist[pl.ds(ci * _L, _L)]
            hist[pl.ds(ci * _L, _L)] = zero16
            local = plsc.cumsum(acc)
            cum = local + carry
            suffix = total - cum + acc
            ge = suffix >= k_cur
            nge_v = nge_v + ge.astype(jnp.int32)
            na_v = na_v + jnp.where(ge, 0, acc)
            carry = carry + local[_L - 1]
        return jnp.sum(nge_v) - 1, jnp.sum(na_v)

    def select_row(row_ref, c):
        @plsc.parallel_loop(0, _N, step=_L, unroll=8)
        def _(j):
            x = row_ref[pl.ds(j, _L)]
            s = plsc.bitcast(x, jnp.int32)
            bucket = (key_of(s) >> 23) + 256
            plsc.addupdate_scatter(hist, [bucket], ones16)

        bstar1, na1 = select_bucket(_NB1, jnp.int32(_K), jnp.int32(_N))
        kc = jnp.int32(_K) - na1
        k_acc = (bstar1 - 256) << 23

        def c1(j, st):
            cnt, sa = st
            x = row_ref[pl.ds(j, _L)]
            s = plsc.bitcast(x, jnp.int32)
            key = key_of(s)
            bucket = (key >> 23) + 256
            gt = bucket > bstar1
            sa = sa + jnp.where(gt, x, 0.0)
            eq = bucket == bstar1
            b2 = (key >> 14) & jnp.int32(511)
            plsc.addupdate_scatter(hist, [b2], ones16, mask=eq)
            plsc.store_scatter(lista, [lane_seg + cnt], key, mask=eq)
            cnt = cnt + eq.astype(jnp.int32)
            return cnt, sa

        cnt1, sa = plsc.parallel_loop(0, _N, step=_L,
                                      carry=(zero16, zero16f),
                                      unroll=8)(c1)

        bstar2, na2 = select_bucket(512, kc, jnp.sum(cnt1))
        lo2 = k_acc + (bstar2 << 14)
        hi2 = lo2 + jnp.int32((1 << 14) - 1)
        kc = kc - na2
        k_acc = lo2

        def c2(j, st):
            cnt, sa = st
            key = plsc.load_gather(lista, [lane_seg + j])
            valid = j < cnt1
            v = plsc.bitcast(key_of(key), jnp.float32)
            gt = jnp.logical_and(key > hi2, valid)
            sa = sa + jnp.where(gt, v, 0.0)
            eq = jnp.logical_and(jnp.logical_and(key >= lo2, key <= hi2),
                                 valid)
            b3 = (key >> 5) & jnp.int32(511)
            plsc.addupdate_scatter(hist, [b3], ones16, mask=eq)
            plsc.store_scatter(listb, [lane_seg + cnt], key, mask=eq)
            cnt = cnt + eq.astype(jnp.int32)
            return cnt, sa

        cnt2, sa = plsc.parallel_loop(0, jnp.max(cnt1), carry=(zero16, sa),
                                      unroll=8)(c2)

        bstar3, na3 = select_bucket(512, kc, jnp.sum(cnt2))
        lo3 = k_acc + (bstar3 << 5)
        hi3 = lo3 + jnp.int32(31)
        kc = kc - na3
        k_acc = lo3

        def c3(j, sa):
            key = plsc.load_gather(listb, [lane_seg + j])
            valid = j < cnt2
            v = plsc.bitcast(key_of(key), jnp.float32)
            gt = jnp.logical_and(key > hi3, valid)
            sa = sa + jnp.where(gt, v, 0.0)
            eq = jnp.logical_and(jnp.logical_and(key >= lo3, key <= hi3),
                                 valid)
            b4 = key & jnp.int32(31)
            plsc.addupdate_scatter(hist, [b4], ones16, mask=eq)
            plsc.addupdate_scatter(sumh, [b4], v, mask=eq)
            return sa

        sa = plsc.parallel_loop(0, jnp.max(cnt2), carry=sa, unroll=8)(c3)

        t4 = hist[pl.ds(0, _L)] + hist[pl.ds(_L, _L)]
        total4 = jnp.sum(t4)
        carry = jnp.int32(0)
        na_v = zero16
        nge_v = zero16
        sa4_v = zero16f
        for ci in range(2):
            acc = hist[pl.ds(ci * _L, _L)]
            hist[pl.ds(ci * _L, _L)] = zero16
            accs = sumh[pl.ds(ci * _L, _L)]
            sumh[pl.ds(ci * _L, _L)] = zero16f
            local = plsc.cumsum(acc)
            cum = local + carry
            suffix = total4 - cum + acc
            ge = suffix >= kc
            nge_v = nge_v + ge.astype(jnp.int32)
            na_v = na_v + jnp.where(ge, 0, acc)
            sa4_v = sa4_v + jnp.where(ge, 0.0, accs)
            carry = carry + local[_L - 1]
        bstar4 = jnp.sum(nge_v) - 1
        kc = kc - jnp.sum(na_v)
        k_acc = k_acc + bstar4

        vk = plsc.bitcast(key_of(bcast_i(k_acc)), jnp.float32)
        ties = jnp.where(lane0, bcast_f(kc.astype(jnp.float32)) * vk, 0.0)
        sal = jnp.sum((sa + sa4_v + ties) * (1.0 / _K))
        plsc.store_scatter(salbuf, [bcast_i(c)], bcast_f(sal), mask=lane0)

    pltpu.make_async_copy(fm_hbm.at[b, 0], row0, sem0).start()
    for ci in range(_HS // _L):
        hist[pl.ds(ci * _L, _L)] = zero16
    sumh[pl.ds(0, _L)] = zero16f
    sumh[pl.ds(_L, _L)] = zero16f
    pltpu.sync_copy(pooled_hbm.at[b], pooledbuf)
    pltpu.sync_copy(params_hbm, parbuf)

    def rowpair(i, _):
        c0 = i * 2
        pltpu.make_async_copy(fm_hbm.at[b, c0], row0, sem0).wait()
        cp1 = pltpu.make_async_copy(fm_hbm.at[b, c0 + 1], row1, sem1)
        cp1.start()
        select_row(row0, c0)
        cp1.wait()

        @pl.when(i < _C // 2 - 1)
        def _():
            pltpu.make_async_copy(fm_hbm.at[b, c0 + 2], row0, sem0).start()

        select_row(row1, c0 + 1)
        return 0

    lax.fori_loop(0, _C // 2, rowpair, 0)

    def rsqrt16(v):
        i = plsc.bitcast(v, jnp.int32)
        y = plsc.bitcast(jnp.int32(0x5F3759DF) - (i >> 1), jnp.float32)
        for _ in range(4):
            y = y * (1.5 - 0.5 * v * y * y)
        return y

    nch = _C // _L

    def stats(ref):
        accv = zero16f
        for ci in range(nch):
            accv = accv + ref[pl.ds(ci * _L, _L)]
        mean = bcast_f(jnp.sum(accv) * (1.0 / _C))
        varv = zero16f
        for ci in range(nch):
            d = ref[pl.ds(ci * _L, _L)] - mean
            varv = varv + d * d
        rstd = rsqrt16(bcast_f(jnp.sum(varv) * (1.0 / _C)) + _LN_EPS)
        return mean, rstd

    mean1, rstd1 = stats(salbuf)
    for ci in range(nch):
        sl = pl.ds(ci * _L, _L)
        w1 = parbuf[pl.ds(0 * _C + ci * _L, _L)]
        b1 = parbuf[pl.ds(1 * _C + ci * _L, _L)]
        gs = parbuf[pl.ds(2 * _C + ci * _L, _L)]
        gb = parbuf[pl.ds(3 * _C + ci * _L, _L)]
        rs = parbuf[pl.ds(4 * _C + ci * _L, _L)]
        sal = (salbuf[sl] - mean1) * rstd1 * w1 + b1
        pld = pooledbuf[sl]
        gate = 1.0 / (1.0 + jnp.exp(-(gs * sal + gb)))
        outbuf[sl] = pld + rs * gate * (sal - pld)

    mean2, rstd2 = stats(outbuf)
    for ci in range(nch):
        sl = pl.ds(ci * _L, _L)
        w2 = parbuf[pl.ds(5 * _C + ci * _L, _L)]
        b2 = parbuf[pl.ds(6 * _C + ci * _L, _L)]
        outbuf[sl] = (outbuf[sl] - mean2) * rstd2 * w2 + b2

    pltpu.sync_copy(outbuf, out_hbm.at[b])


_sc_kernel = functools.partial(
    pl.kernel,
    mesh=plsc.VectorSubcoreMesh(core_axis_name="c", subcore_axis_name="s"),
    out_type=jax.ShapeDtypeStruct((_B, _C), jnp.float32),
    compiler_params=pltpu.CompilerParams(needs_layout_passes=False),
    scratch_types=[
        pltpu.VMEM((_N,), jnp.float32),
        pltpu.VMEM((_N,), jnp.float32),
        pltpu.VMEM((_N,), jnp.int32),
        pltpu.VMEM((_N,), jnp.int32),
        pltpu.VMEM((_HS,), jnp.int32),
        pltpu.VMEM((2 * _L,), jnp.float32),
        pltpu.VMEM((_C,), jnp.float32),
        pltpu.VMEM((_C,), jnp.float32),
        pltpu.VMEM((7 * _C,), jnp.float32),
        pltpu.VMEM((_C,), jnp.float32),
        pltpu.SemaphoreType.DMA,
        pltpu.SemaphoreType.DMA,
    ],
)(_sc_body)


def kernel(pooled, feature_map, ln1_w, ln1_b, gate_scale, gate_bias,
           residual_scale, ln2_w, ln2_b):
    fm = feature_map.astype(jnp.float32).reshape(_B, _C, _N)
    params = jnp.concatenate([
        ln1_w, ln1_b, gate_scale, gate_bias, residual_scale, ln2_w, ln2_b,
    ]).astype(jnp.float32)
    out = _sc_kernel(pooled.astype(jnp.float32), fm, params)
    return out.astype(pooled.dtype)


# speedup vs baseline: 1.6443x; 1.1386x over previous
import functools

import jax
import jax.numpy as jnp
from jax import lax
from jax.experimental import pallas as pl
from jax.experimental.pallas import tpu as pltpu
from jax.experimental.pallas import tpu_sc as plsc

_LN_EPS = 1e-5
_B, _C, _N = 32, 96, 16384
_K = 4096
_L = 16
_SEG = _N // _L
_NB1 = 1024
_HS = 1024
_LEVELS = [(14, 512), (5, 512), (0, 32)]
_NCORES = 2


def _sc_body(pooled_hbm, fm_hbm, params_hbm, out_hbm,
             row0, row1, lista, listb, hist, sumh,
             salbuf, pooledbuf, parbuf, outbuf, sem0, sem1):
    iota = lax.iota(jnp.int32, _L)
    lane_seg = iota * _SEG
    zero16 = jnp.zeros((_L,), jnp.int32)
    zero16f = jnp.zeros((_L,), jnp.float32)
    ones16 = jnp.ones((_L,), jnp.int32)
    lane0 = iota == 0

    b = lax.axis_index("s") * _NCORES + lax.axis_index("c")

    def bcast_i(x):
        return lax.broadcast_in_dim(x, (_L,), ())

    def bcast_f(x):
        return lax.broadcast_in_dim(x, (_L,), ())

    def key_of(s):
        return s ^ ((s >> 31) & jnp.int32(0x7FFFFFFF))

    def select_bucket(nb, k_cur, total):
        carry = jnp.int32(0)
        na_v = zero16
        nge_v = zero16
        for ci in range(nb // _L):
            acc = hist[pl.ds(ci * _L, _L)]
            hist[pl.ds(ci * _L, _L)] = zero16
            local = plsc.cumsum(acc)
            cum = local + carry
            suffix = total - cum + acc
            ge = suffix >= k_cur
            nge_v = nge_v + ge.astype(jnp.int32)
            na_v = na_v + jnp.where(ge, 0, acc)
            carry = carry + local[_L - 1]
        return jnp.sum(nge_v) - 1, jnp.sum(na_v)

    def select_row(row_ref, c):
        @plsc.parallel_loop(0, _N, step=_L, unroll=8)
        def _(j):
            x = row_ref[pl.ds(j, _L)]
            s = plsc.bitcast(x, jnp.int32)
            bucket = (key_of(s) >> 22) + 512
            plsc.addupdate_scatter(hist, [bucket], ones16)

        bstar1, na1 = select_bucket(_NB1, jnp.int32(_K), jnp.int32(_N))
        kc = jnp.int32(_K) - na1
        k_acc = (bstar1 - 512) << 22

        def c1(j, st):
            cnt, sa = st
            x = row_ref[pl.ds(j, _L)]
            s = plsc.bitcast(x, jnp.int32)
            key = key_of(s)
            bucket = (key >> 22) + 512
            gt = bucket > bstar1
            sa = sa + jnp.where(gt, x, 0.0)
            eq = bucket == bstar1
            b2 = (key >> 13) & jnp.int32(511)
            plsc.addupdate_scatter(hist, [b2], ones16, mask=eq)
            plsc.store_scatter(lista, [lane_seg + cnt], key, mask=eq)
            cnt = cnt + eq.astype(jnp.int32)
            return cnt, sa

        cnt1, sa = plsc.parallel_loop(0, _N, step=_L,
                                      carry=(zero16, zero16f),
                                      unroll=8)(c1)

        bstar2, na2 = select_bucket(512, kc, jnp.sum(cnt1))
        lo2 = k_acc + (bstar2 << 13)
        hi2 = lo2 + jnp.int32((1 << 13) - 1)
        kc = kc - na2
        k_acc = lo2

        def c2(j, st):
            cnt, sa = st
            key = plsc.load_gather(lista, [lane_seg + j])
            valid = j < cnt1
            v = plsc.bitcast(key_of(key), jnp.float32)
            gt = jnp.logical_and(key > hi2, valid)
            sa = sa + jnp.where(gt, v, 0.0)
            eq = jnp.logical_and(jnp.logical_and(key >= lo2, key <= hi2),
                                 valid)
            b3 = (key >> 4) & jnp.int32(511)
            plsc.addupdate_scatter(hist, [b3], ones16, mask=eq)
            plsc.store_scatter(listb, [lane_seg + cnt], key, mask=eq)
            cnt = cnt + eq.astype(jnp.int32)
            return cnt, sa

        cnt2, sa = plsc.parallel_loop(0, jnp.max(cnt1), carry=(zero16, sa),
                                      unroll=8)(c2)

        bstar3, na3 = select_bucket(512, kc, jnp.sum(cnt2))
        lo3 = k_acc + (bstar3 << 4)
        hi3 = lo3 + jnp.int32(15)
        kc = kc - na3
        k_acc = lo3

        def c3(j, sa):
            key = plsc.load_gather(listb, [lane_seg + j])
            valid = j < cnt2
            v = plsc.bitcast(key_of(key), jnp.float32)
            gt = jnp.logical_and(key > hi3, valid)
            sa = sa + jnp.where(gt, v, 0.0)
            eq = jnp.logical_and(jnp.logical_and(key >= lo3, key <= hi3),
                                 valid)
            b4 = key & jnp.int32(15)
            plsc.addupdate_scatter(hist, [b4], ones16, mask=eq)
            plsc.addupdate_scatter(sumh, [b4], v, mask=eq)
            return sa

        sa = plsc.parallel_loop(0, jnp.max(cnt2), carry=sa, unroll=8)(c3)

        t4 = hist[pl.ds(0, _L)]
        total4 = jnp.sum(t4)
        carry = jnp.int32(0)
        na_v = zero16
        nge_v = zero16
        sa4_v = zero16f
        for ci in range(1):
            acc = hist[pl.ds(ci * _L, _L)]
            hist[pl.ds(ci * _L, _L)] = zero16
            accs = sumh[pl.ds(ci * _L, _L)]
            sumh[pl.ds(ci * _L, _L)] = zero16f
            local = plsc.cumsum(acc)
            cum = local + carry
            suffix = total4 - cum + acc
            ge = suffix >= kc
            nge_v = nge_v + ge.astype(jnp.int32)
            na_v = na_v + jnp.where(ge, 0, acc)
            sa4_v = sa4_v + jnp.where(ge, 0.0, accs)
            carry = carry + local[_L - 1]
        bstar4 = jnp.sum(nge_v) - 1
        kc = kc - jnp.sum(na_v)
        k_acc = k_acc + bstar4

        vk = plsc.bitcast(key_of(bcast_i(k_acc)), jnp.float32)
        ties = jnp.where(lane0, bcast_f(kc.astype(jnp.float32)) * vk, 0.0)
        sal = jnp.sum((sa + sa4_v + ties) * (1.0 / _K))
        plsc.store_scatter(salbuf, [bcast_i(c)], bcast_f(sal), mask=lane0)

    pltpu.make_async_copy(fm_hbm.at[b, 0], row0, sem0).start()
    for ci in range(_HS // _L):
        hist[pl.ds(ci * _L, _L)] = zero16
    sumh[pl.ds(0, _L)] = zero16f
    sumh[pl.ds(_L, _L)] = zero16f
    pltpu.sync_copy(pooled_hbm.at[b], pooledbuf)
    pltpu.sync_copy(params_hbm, parbuf)

    def rowpair(i, _):
        c0 = i * 2
        pltpu.make_async_copy(fm_hbm.at[b, c0], row0, sem0).wait()
        cp1 = pltpu.make_async_copy(fm_hbm.at[b, c0 + 1], row1, sem1)
        cp1.start()
        select_row(row0, c0)
        cp1.wait()

        @pl.when(i < _C // 2 - 1)
        def _():
            pltpu.make_async_copy(fm_hbm.at[b, c0 + 2], row0, sem0).start()

        select_row(row1, c0 + 1)
        return 0

    lax.fori_loop(0, _C // 2, rowpair, 0)

    def rsqrt16(v):
        i = plsc.bitcast(v, jnp.int32)
        y = plsc.bitcast(jnp.int32(0x5F3759DF) - (i >> 1), jnp.float32)
        for _ in range(4):
            y = y * (1.5 - 0.5 * v * y * y)
        return y

    nch = _C // _L

    def stats(ref):
        accv = zero16f
        for ci in range(nch):
            accv = accv + ref[pl.ds(ci * _L, _L)]
        mean = bcast_f(jnp.sum(accv) * (1.0 / _C))
        varv = zero16f
        for ci in range(nch):
            d = ref[pl.ds(ci * _L, _L)] - mean
            varv = varv + d * d
        rstd = rsqrt16(bcast_f(jnp.sum(varv) * (1.0 / _C)) + _LN_EPS)
        return mean, rstd

    mean1, rstd1 = stats(salbuf)
    for ci in range(nch):
        sl = pl.ds(ci * _L, _L)
        w1 = parbuf[pl.ds(0 * _C + ci * _L, _L)]
        b1 = parbuf[pl.ds(1 * _C + ci * _L, _L)]
        gs = parbuf[pl.ds(2 * _C + ci * _L, _L)]
        gb = parbuf[pl.ds(3 * _C + ci * _L, _L)]
        rs = parbuf[pl.ds(4 * _C + ci * _L, _L)]
        sal = (salbuf[sl] - mean1) * rstd1 * w1 + b1
        pld = pooledbuf[sl]
        gate = 1.0 / (1.0 + jnp.exp(-(gs * sal + gb)))
        outbuf[sl] = pld + rs * gate * (sal - pld)

    mean2, rstd2 = stats(outbuf)
    for ci in range(nch):
        sl = pl.ds(ci * _L, _L)
        w2 = parbuf[pl.ds(5 * _C + ci * _L, _L)]
        b2 = parbuf[pl.ds(6 * _C + ci * _L, _L)]
        outbuf[sl] = (outbuf[sl] - mean2) * rstd2 * w2 + b2

    pltpu.sync_copy(outbuf, out_hbm.at[b])


_sc_kernel = functools.partial(
    pl.kernel,
    mesh=plsc.VectorSubcoreMesh(core_axis_name="c", subcore_axis_name="s"),
    out_type=jax.ShapeDtypeStruct((_B, _C), jnp.float32),
    compiler_params=pltpu.CompilerParams(needs_layout_passes=False),
    scratch_types=[
        pltpu.VMEM((_N,), jnp.float32),
        pltpu.VMEM((_N,), jnp.float32),
        pltpu.VMEM((_N,), jnp.int32),
        pltpu.VMEM((_N,), jnp.int32),
        pltpu.VMEM((_HS,), jnp.int32),
        pltpu.VMEM((2 * _L,), jnp.float32),
        pltpu.VMEM((_C,), jnp.float32),
        pltpu.VMEM((_C,), jnp.float32),
        pltpu.VMEM((7 * _C,), jnp.float32),
        pltpu.VMEM((_C,), jnp.float32),
        pltpu.SemaphoreType.DMA,
        pltpu.SemaphoreType.DMA,
    ],
)(_sc_body)


def kernel(pooled, feature_map, ln1_w, ln1_b, gate_scale, gate_bias,
           residual_scale, ln2_w, ln2_b):
    fm = feature_map.astype(jnp.float32).reshape(_B, _C, _N)
    params = jnp.concatenate([
        ln1_w, ln1_b, gate_scale, gate_bias, residual_scale, ln2_w, ln2_b,
    ]).astype(jnp.float32)
    out = _sc_kernel(pooled.astype(jnp.float32), fm, params)
    return out.astype(pooled.dtype)


# speedup vs baseline: 1.6563x; 1.0073x over previous
import functools

import jax
import jax.numpy as jnp
from jax import lax
from jax.experimental import pallas as pl
from jax.experimental.pallas import tpu as pltpu
from jax.experimental.pallas import tpu_sc as plsc

_LN_EPS = 1e-5
_B, _C, _N = 32, 96, 16384
_K = 4096
_L = 16
_SEG = _N // _L
_NB1 = 2048
_HS = 2048
_LEVELS = [(14, 512), (5, 512), (0, 32)]
_NCORES = 2


def _sc_body(pooled_hbm, fm_hbm, params_hbm, out_hbm,
             row0, row1, lista, listb, hist, sumh,
             salbuf, pooledbuf, parbuf, outbuf, sem0, sem1):
    iota = lax.iota(jnp.int32, _L)
    lane_seg = iota * _SEG
    zero16 = jnp.zeros((_L,), jnp.int32)
    zero16f = jnp.zeros((_L,), jnp.float32)
    ones16 = jnp.ones((_L,), jnp.int32)
    lane0 = iota == 0

    b = lax.axis_index("s") * _NCORES + lax.axis_index("c")

    def bcast_i(x):
        return lax.broadcast_in_dim(x, (_L,), ())

    def bcast_f(x):
        return lax.broadcast_in_dim(x, (_L,), ())

    def key_of(s):
        return s ^ ((s >> 31) & jnp.int32(0x7FFFFFFF))

    def select_bucket(nb, k_cur, total):
        carry = jnp.int32(0)
        na_v = zero16
        nge_v = zero16
        for ci in range(nb // _L):
            acc = hist[pl.ds(ci * _L, _L)]
            hist[pl.ds(ci * _L, _L)] = zero16
            local = plsc.cumsum(acc)
            cum = local + carry
            suffix = total - cum + acc
            ge = suffix >= k_cur
            nge_v = nge_v + ge.astype(jnp.int32)
            na_v = na_v + jnp.where(ge, 0, acc)
            carry = carry + local[_L - 1]
        return jnp.sum(nge_v) - 1, jnp.sum(na_v)

    def select_row(row_ref, c):
        @plsc.parallel_loop(0, _N, step=_L, unroll=8)
        def _(j):
            x = row_ref[pl.ds(j, _L)]
            s = plsc.bitcast(x, jnp.int32)
            bucket = (key_of(s) >> 21) + 1024
            plsc.addupdate_scatter(hist, [bucket], ones16)

        bstar1, na1 = select_bucket(_NB1, jnp.int32(_K), jnp.int32(_N))
        kc = jnp.int32(_K) - na1
        k_acc = (bstar1 - 1024) << 21

        def c1(j, st):
            cnt, sa = st
            x = row_ref[pl.ds(j, _L)]
            s = plsc.bitcast(x, jnp.int32)
            key = key_of(s)
            bucket = (key >> 21) + 1024
            gt = bucket > bstar1
            sa = sa + jnp.where(gt, x, 0.0)
            eq = bucket == bstar1
            b2 = (key >> 12) & jnp.int32(511)
            plsc.addupdate_scatter(hist, [b2], ones16, mask=eq)
            plsc.store_scatter(lista, [lane_seg + cnt], key, mask=eq)
            cnt = cnt + eq.astype(jnp.int32)
            return cnt, sa

        cnt1, sa = plsc.parallel_loop(0, _N, step=_L,
                                      carry=(zero16, zero16f),
                                      unroll=8)(c1)

        bstar2, na2 = select_bucket(512, kc, jnp.sum(cnt1))
        lo2 = k_acc + (bstar2 << 12)
        hi2 = lo2 + jnp.int32((1 << 12) - 1)
        kc = kc - na2
        k_acc = lo2

        def c2(j, st):
            cnt, sa = st
            key = plsc.load_gather(lista, [lane_seg + j])
            valid = j < cnt1
            v = plsc.bitcast(key_of(key), jnp.float32)
            gt = jnp.logical_and(key > hi2, valid)
            sa = sa + jnp.where(gt, v, 0.0)
            eq = jnp.logical_and(jnp.logical_and(key >= lo2, key <= hi2),
                                 valid)
            b3 = (key >> 3) & jnp.int32(511)
            plsc.addupdate_scatter(hist, [b3], ones16, mask=eq)
            plsc.store_scatter(listb, [lane_seg + cnt], key, mask=eq)
            cnt = cnt + eq.astype(jnp.int32)
            return cnt, sa

        cnt2, sa = plsc.parallel_loop(0, jnp.max(cnt1), carry=(zero16, sa),
                                      unroll=8)(c2)

        bstar3, na3 = select_bucket(512, kc, jnp.sum(cnt2))
        lo3 = k_acc + (bstar3 << 3)
        hi3 = lo3 + jnp.int32(7)
        kc = kc - na3
        k_acc = lo3

        def c3(j, sa):
            key = plsc.load_gather(listb, [lane_seg + j])
            valid = j < cnt2
            v = plsc.bitcast(key_of(key), jnp.float32)
            gt = jnp.logical_and(key > hi3, valid)
            sa = sa + jnp.where(gt, v, 0.0)
            eq = jnp.logical_and(jnp.logical_and(key >= lo3, key <= hi3),
                                 valid)
            b4 = key & jnp.int32(7)
            plsc.addupdate_scatter(hist, [b4], ones16, mask=eq)
            plsc.addupdate_scatter(sumh, [b4], v, mask=eq)
            return sa

        sa = plsc.parallel_loop(0, jnp.max(cnt2), carry=sa, unroll=8)(c3)

        t4 = hist[pl.ds(0, _L)]
        total4 = jnp.sum(t4)
        carry = jnp.int32(0)
        na_v = zero16
        nge_v = zero16
        sa4_v = zero16f
        for ci in range(1):
            acc = hist[pl.ds(ci * _L, _L)]
            hist[pl.ds(ci * _L, _L)] = zero16
            accs = sumh[pl.ds(ci * _L, _L)]
            sumh[pl.ds(ci * _L, _L)] = zero16f
            local = plsc.cumsum(acc)
            cum = local + carry
            suffix = total4 - cum + acc
            ge = suffix >= kc
            nge_v = nge_v + ge.astype(jnp.int32)
            na_v = na_v + jnp.where(ge, 0, acc)
            sa4_v = sa4_v + jnp.where(ge, 0.0, accs)
            carry = carry + local[_L - 1]
        bstar4 = jnp.sum(nge_v) - 1
        kc = kc - jnp.sum(na_v)
        k_acc = k_acc + bstar4

        vk = plsc.bitcast(key_of(bcast_i(k_acc)), jnp.float32)
        ties = jnp.where(lane0, bcast_f(kc.astype(jnp.float32)) * vk, 0.0)
        sal = jnp.sum((sa + sa4_v + ties) * (1.0 / _K))
        plsc.store_scatter(salbuf, [bcast_i(c)], bcast_f(sal), mask=lane0)

    pltpu.make_async_copy(fm_hbm.at[b, 0], row0, sem0).start()
    for ci in range(_HS // _L):
        hist[pl.ds(ci * _L, _L)] = zero16
    sumh[pl.ds(0, _L)] = zero16f
    sumh[pl.ds(_L, _L)] = zero16f
    pltpu.sync_copy(pooled_hbm.at[b], pooledbuf)
    pltpu.sync_copy(params_hbm, parbuf)

    def rowpair(i, _):
        c0 = i * 2
        pltpu.make_async_copy(fm_hbm.at[b, c0], row0, sem0).wait()
        cp1 = pltpu.make_async_copy(fm_hbm.at[b, c0 + 1], row1, sem1)
        cp1.start()
        select_row(row0, c0)
        cp1.wait()

        @pl.when(i < _C // 2 - 1)
        def _():
            pltpu.make_async_copy(fm_hbm.at[b, c0 + 2], row0, sem0).start()

        select_row(row1, c0 + 1)
        return 0

    lax.fori_loop(0, _C // 2, rowpair, 0)

    def rsqrt16(v):
        i = plsc.bitcast(v, jnp.int32)
        y = plsc.bitcast(jnp.int32(0x5F3759DF) - (i >> 1), jnp.float32)
        for _ in range(4):
            y = y * (1.5 - 0.5 * v * y * y)
        return y

    nch = _C // _L

    def stats(ref):
        accv = zero16f
        for ci in range(nch):
            accv = accv + ref[pl.ds(ci * _L, _L)]
        mean = bcast_f(jnp.sum(accv) * (1.0 / _C))
        varv = zero16f
        for ci in range(nch):
            d = ref[pl.ds(ci * _L, _L)] - mean
            varv = varv + d * d
        rstd = rsqrt16(bcast_f(jnp.sum(varv) * (1.0 / _C)) + _LN_EPS)
        return mean, rstd

    mean1, rstd1 = stats(salbuf)
    for ci in range(nch):
        sl = pl.ds(ci * _L, _L)
        w1 = parbuf[pl.ds(0 * _C + ci * _L, _L)]
        b1 = parbuf[pl.ds(1 * _C + ci * _L, _L)]
        gs = parbuf[pl.ds(2 * _C + ci * _L, _L)]
        gb = parbuf[pl.ds(3 * _C + ci * _L, _L)]
        rs = parbuf[pl.ds(4 * _C + ci * _L, _L)]
        sal = (salbuf[sl] - mean1) * rstd1 * w1 + b1
        pld = pooledbuf[sl]
        gate = 1.0 / (1.0 + jnp.exp(-(gs * sal + gb)))
        outbuf[sl] = pld + rs * gate * (sal - pld)

    mean2, rstd2 = stats(outbuf)
    for ci in range(nch):
        sl = pl.ds(ci * _L, _L)
        w2 = parbuf[pl.ds(5 * _C + ci * _L, _L)]
        b2 = parbuf[pl.ds(6 * _C + ci * _L, _L)]
        outbuf[sl] = (outbuf[sl] - mean2) * rstd2 * w2 + b2

    pltpu.sync_copy(outbuf, out_hbm.at[b])


_sc_kernel = functools.partial(
    pl.kernel,
    mesh=plsc.VectorSubcoreMesh(core_axis_name="c", subcore_axis_name="s"),
    out_type=jax.ShapeDtypeStruct((_B, _C), jnp.float32),
    compiler_params=pltpu.CompilerParams(needs_layout_passes=False),
    scratch_types=[
        pltpu.VMEM((_N,), jnp.float32),
        pltpu.VMEM((_N,), jnp.float32),
        pltpu.VMEM((_N,), jnp.int32),
        pltpu.VMEM((_N,), jnp.int32),
        pltpu.VMEM((_HS,), jnp.int32),
        pltpu.VMEM((2 * _L,), jnp.float32),
        pltpu.VMEM((_C,), jnp.float32),
        pltpu.VMEM((_C,), jnp.float32),
        pltpu.VMEM((7 * _C,), jnp.float32),
        pltpu.VMEM((_C,), jnp.float32),
        pltpu.SemaphoreType.DMA,
        pltpu.SemaphoreType.DMA,
    ],
)(_sc_body)


def kernel(pooled, feature_map, ln1_w, ln1_b, gate_scale, gate_bias,
           residual_scale, ln2_w, ln2_b):
    fm = feature_map.astype(jnp.float32).reshape(_B, _C, _N)
    params = jnp.concatenate([
        ln1_w, ln1_b, gate_scale, gate_bias, residual_scale, ln2_w, ln2_b,
    ]).astype(jnp.float32)
    out = _sc_kernel(pooled.astype(jnp.float32), fm, params)
    return out.astype(pooled.dtype)
